# baseline (speedup 1.0000x reference)
import jax
import jax.numpy as jnp
from jax import lax
from jax.experimental import pallas as pl
from jax.experimental.pallas import tpu as pltpu
from jax.experimental.pallas import tpu_sc as plsc

_D = 32
_DT = 4
_S = 50
_B = 16384
_BR = 128
_NBT = _B // _BR
_NC = 2
_NS = 16
_NW = _NC * _NS
_BT_PER_W = _NBT // _NW
_SB = 5
_NSB = _BT_PER_W * _S // _SB


def _emb_lookup(idx_hbm, table_hbm, out_hbm, idxraw_v, idxs_v, a_v, b_v,
                gsem, ssem):
    wid = lax.axis_index("s") * _NC + lax.axis_index("c")
    iota = lax.iota(jnp.int32, 16)
    row16 = [iota + 16 * g for g in range(8)]
    row50 = [(iota + 16 * g) * _S for g in range(8)]

    for bti in range(_BT_PER_W):
        base = (wid * _BT_PER_W + bti) * _BR * _S
        pltpu.sync_copy(idx_hbm.at[pl.ds(base, _BR * _S)], idxraw_v)

        def tbody(s, carry, bti=bti):
            for g in range(8):
                v = plsc.load_gather(idxraw_v, [row50[g] + s])
                idxs_v[bti, pl.ds(s * _BR + 16 * g, 16)] = v
            return carry

        lax.fori_loop(0, _S, tbody, 0)

    def fire_gather(sb, r):
        bti = sb // 10
        s0 = (sb % 10) * _SB
        pltpu.async_copy(
            table_hbm.at[idxs_v.at[bti, pl.ds(s0 * _BR, _SB * _BR)]],
            a_v.at[r], gsem,
        )

    def wait_gather(sb, r):
        bti = sb // 10
        s0 = (sb % 10) * _SB
        pltpu.make_async_copy(
            table_hbm.at[idxs_v.at[bti, pl.ds(s0 * _BR, _SB * _BR)]],
            a_v.at[r], gsem,
        ).wait()

    def out_tiles(sb, r, fire):
        bt = wid * _BT_PER_W + sb // 10
        s0 = (sb % 10) * _SB
        for k in range(_SB):
            for dt in range(_DT):
                cp = (pltpu.async_copy if fire else
                      lambda s, d, m: pltpu.make_async_copy(s, d, m).wait())
                cp(b_v.at[r, k, dt], out_hbm.at[s0 + k, dt, bt], ssem)

    fire_gather(0, 0)
    fire_gather(1, 1)

    def body(g2, carry):
        for r in range(2):
            sb = 2 * g2 + r
            wait_gather(sb, r)

            @pl.when(g2 >= 1)
            def _():
                out_tiles(sb - 2, r, fire=False)

            def dbody(d0, carry2, r=r):
                cmod = (iota + d0) & 31
                dtv = cmod >> 3
                drv = cmod & 7
                for k in range(_SB):
                    for g in range(8):
                        v = plsc.load_gather(
                            a_v.at[r], [row16[g] + k * _BR, cmod])
                        plsc.store_scatter(
                            b_v.at[r, k], [dtv, drv, row16[g]], v)
                return carry2

            lax.fori_loop(0, _D, dbody, 0)

            out_tiles(sb, r, fire=True)

            @pl.when(g2 < (_NSB - 2) // 2)
            def _():
                fire_gather(sb + 2, r)
        return carry

    lax.fori_loop(0, _NSB // 2, body, 0)

    for r in range(2):
        out_tiles(_NSB - 2 + r, r, fire=False)


def kernel(inputs, word_embeddings):
    idx_flat = inputs.reshape(_B * _S).astype(jnp.int32) & 0xFFFFF
    out5 = pl.kernel(
        _emb_lookup,
        out_type=jax.ShapeDtypeStruct((_S, _DT, _NBT, 8, _BR), jnp.float32),
        mesh=plsc.VectorSubcoreMesh(core_axis_name="c", subcore_axis_name="s"),
        scratch_types=[
            pltpu.VMEM((_BR * _S,), jnp.int32),
            pltpu.VMEM((_BT_PER_W, _S * _BR), jnp.int32),
            pltpu.VMEM((2, _SB * _BR, _D), jnp.float32),
            pltpu.VMEM((2, _SB, _DT, 8, _BR), jnp.float32),
            pltpu.SemaphoreType.DMA,
            pltpu.SemaphoreType.DMA,
        ],
        compiler_params=pltpu.CompilerParams(
            use_tc_tiling_on_sc=False, needs_layout_passes=False
        ),
    )(idx_flat, word_embeddings)
    return out5.transpose(2, 4, 0, 1, 3).reshape(_B, _S, _D)


# speedup vs baseline: 1.0004x; 1.0004x over previous
import jax
import jax.numpy as jnp
from jax import lax
from jax.experimental import pallas as pl
from jax.experimental.pallas import tpu as pltpu
from jax.experimental.pallas import tpu_sc as plsc

_D = 32
_DT = 4
_S = 50
_B = 16384
_BR = 128
_NBT = _B // _BR
_NC = 2
_NS = 16
_NW = _NC * _NS
_BT_PER_W = _NBT // _NW
_SB = 5
_NSB = _BT_PER_W * _S // _SB


def _emb_lookup(idx_hbm, table_hbm, out_hbm, idxraw_v, idxs_v, a_v, b_v,
                gsem, ssem):
    wid = lax.axis_index("s") * _NC + lax.axis_index("c")
    iota = lax.iota(jnp.int32, 16)
    row16 = [iota + 16 * g for g in range(8)]
    row50 = [(iota + 16 * g) * _S for g in range(8)]

    for bti in range(_BT_PER_W):
        base = (wid * _BT_PER_W + bti) * _BR * _S
        pltpu.sync_copy(idx_hbm.at[pl.ds(base, _BR * _S)], idxraw_v)

        def tbody(s, carry, bti=bti):
            for g in range(8):
                v = plsc.load_gather(idxraw_v, [row50[g] + s])
                idxs_v[bti, pl.ds(s * _BR + 16 * g, 16)] = v
            return carry

        lax.fori_loop(0, _S, tbody, 0)

    def fire_gather(sb, r):
        bti = sb // 10
        s0 = (sb % 10) * _SB
        pltpu.async_copy(
            table_hbm.at[idxs_v.at[bti, pl.ds(s0 * _BR, _SB * _BR)]],
            a_v.at[r], gsem,
        )

    def wait_gather(sb, r):
        bti = sb // 10
        s0 = (sb % 10) * _SB
        pltpu.make_async_copy(
            table_hbm.at[idxs_v.at[bti, pl.ds(s0 * _BR, _SB * _BR)]],
            a_v.at[r], gsem,
        ).wait()

    def out_tiles(sb, r, fire):
        bt = wid * _BT_PER_W + sb // 10
        s0 = (sb % 10) * _SB
        for k in range(_SB):
            for dt in range(_DT):
                cp = (pltpu.async_copy if fire else
                      lambda s, d, m: pltpu.make_async_copy(s, d, m).wait())
                cp(b_v.at[r, k, dt], out_hbm.at[s0 + k, dt, bt], ssem)

    fire_gather(0, 0)
    fire_gather(1, 1)

    def body(g2, carry):
        for r in range(2):
            sb = 2 * g2 + r
            wait_gather(sb, r)

            @pl.when(g2 >= 1)
            def _():
                out_tiles(sb - 2, r, fire=False)

            def dbody(d0, carry2, r=r):
                cmod = (iota + d0) & 31
                dtv = cmod >> 3
                drv = cmod & 7
                for k in range(_SB):
                    ak = a_v.at[r, pl.ds(k * _BR, _BR)]
                    for g in range(8):
                        v = plsc.load_gather(ak, [row16[g], cmod])
                        plsc.store_scatter(
                            b_v.at[r, k], [dtv, drv, row16[g]], v)
                return carry2

            lax.fori_loop(0, _D, dbody, 0)

            out_tiles(sb, r, fire=True)

            @pl.when(g2 < (_NSB - 2) // 2)
            def _():
                fire_gather(sb + 2, r)
        return carry

    lax.fori_loop(0, _NSB // 2, body, 0)

    for r in range(2):
        out_tiles(_NSB - 2 + r, r, fire=False)


def kernel(inputs, word_embeddings):
    idx_flat = inputs.reshape(_B * _S).astype(jnp.int32) & 0xFFFFF
    out5 = pl.kernel(
        _emb_lookup,
        out_type=jax.ShapeDtypeStruct((_S, _DT, _NBT, 8, _BR), jnp.float32),
        mesh=plsc.VectorSubcoreMesh(core_axis_name="c", subcore_axis_name="s"),
        scratch_types=[
            pltpu.VMEM((_BR * _S,), jnp.int32),
            pltpu.VMEM((_BT_PER_W, _S * _BR), jnp.int32),
            pltpu.VMEM((2, _SB * _BR, _D), jnp.float32),
            pltpu.VMEM((2, _SB, _DT, 8, _BR), jnp.float32),
            pltpu.SemaphoreType.DMA,
            pltpu.SemaphoreType.DMA,
        ],
        compiler_params=pltpu.CompilerParams(
            use_tc_tiling_on_sc=False, needs_layout_passes=False
        ),
    )(idx_flat, word_embeddings)
    return out5.transpose(2, 4, 0, 1, 3).reshape(_B, _S, _D)
